# baseline (speedup 1.0000x reference)
import functools

import jax
import jax.numpy as jnp
from jax import lax
from jax.experimental import pallas as pl
from jax.experimental.pallas import tpu as pltpu
from jax.experimental.pallas import tpu_sc as plsc

B = 4096
F = 26
V = 1000
E = 128
D1 = 1024
D2 = 512

NSPLIT = 1
BH = B // NSPLIT
NUM_WORKERS = 32
SPW = BH // NUM_WORKERS
RPW = SPW * F
SPAD = 32
NBUF = 4
BLK = 512


def _sc_gather(table_hbm, idx2_hbm, idxs_hbm, wf_hbm, xv2_hbm,
               e2_out, s_out,
               idx_v, idxs_v, wf_v, s_v, xv_v,
               rows0, rows1, brows0, brows1,
               gsem0, gsem1, ssem0, ssem1):
    wid = lax.axis_index("s") * 2 + lax.axis_index("c")
    base = wid * SPW
    rbase = wid * RPW

    rows = (rows0, rows1)
    brows = (brows0, brows1)
    gsem = (gsem0, gsem1)
    ssem = (ssem0, ssem1)
    gcp = [None, None]
    scp = [None, None]

    pltpu.sync_copy(idx2_hbm.at[pl.ds(rbase, RPW)], idx_v)

    def start_gather(c, p):
        idx_c = idx_v.at[pl.ds(c * SPW, SPW)]
        return pltpu.async_copy(table_hbm.at[idx_c], rows[p], gsem[p])

    gcp[0] = start_gather(0, 0)
    gcp[1] = start_gather(1, 1)

    pltpu.sync_copy(xv2_hbm.at[pl.ds(rbase, RPW)], xv_v)
    pltpu.sync_copy(idxs_hbm.at[pl.ds(base * SPAD, SPW * SPAD)], idxs_v)
    pltpu.sync_copy(wf_hbm, wf_v)

    def s_body(i, carry):
        ids = idxs_v[pl.ds(i * 16, 16)]
        vals = plsc.load_gather(wf_v, [ids])
        s_v[pl.ds(i * 16, 16)] = vals
        return carry

    lax.fori_loop(0, SPW * SPAD // 16, s_body, 0)
    pltpu.sync_copy(s_v, s_out.at[pl.ds(base * SPAD, SPW * SPAD)])

    for c in range(F):
        p = c % 2
        gcp[p].wait()
        buf = rows[p]
        bbuf = brows[p]

        def scale_row(r, carry, buf=buf, bbuf=bbuf, c=c):
            bc = plsc.load_gather(
                xv_v, [jnp.full((16,), c * SPW + r, jnp.int32)])
            for k in range(E // 32):
                v0 = buf[r, pl.ds(k * 32, 16)] * bc
                v1 = buf[r, pl.ds(k * 32 + 16, 16)] * bc
                bbuf[r, pl.ds(k * 32, 32)] = plsc.pack(
                    v0, v1, format=plsc.PackFormat.INTERLEAVED)
            return carry

        lax.fori_loop(0, SPW, scale_row, 0)
        if c + 2 < F:
            gcp[p] = start_gather(c + 2, p)
        if scp[p] is not None:
            scp[p].wait()
        scp[p] = pltpu.async_copy(
            bbuf, e2_out.at[pl.ds(base, SPW), pl.ds(c * E, E)], ssem[p])
    for p in range(2):
        if scp[p] is not None:
            scp[p].wait()


def _mlp_body(e2_ref, xv_ref, s_ref, w1s_ref, w1m_ref,
              b1_ref, w2_ref, b2_ref, w3_ref, b3_ref, out_ref):
    scaled = e2_ref[:]
    acc = jnp.dot(scaled, w1s_ref[:], preferred_element_type=jnp.float32)
    acc += jnp.dot((s_ref[:] * xv_ref[:]).astype(jnp.bfloat16), w1m_ref[:],
                   preferred_element_type=jnp.float32)
    h1 = jnp.tanh(acc + b1_ref[:])
    h2 = jnp.tanh(jnp.dot(h1.astype(jnp.bfloat16), w2_ref[:],
                          preferred_element_type=jnp.float32) + b2_ref[:])
    out_ref[:] = (jnp.dot(h2.astype(jnp.bfloat16), w3_ref[:],
                          preferred_element_type=jnp.float32) + b3_ref[0, 0])


def kernel(Xi, Xv, fm_bias, W_first, W_second, W1, b1, W2, b2, W3, b3):
    idx = Xi[:, :, 0].astype(jnp.int32)
    offs = (jnp.arange(F, dtype=jnp.int32) * V)[None, :]
    idx2 = ((idx + offs).reshape(NSPLIT, NUM_WORKERS, SPW, F)
            .transpose(0, 1, 3, 2).reshape(NSPLIT, BH * F))
    xv2 = (Xv.reshape(NSPLIT, NUM_WORKERS, SPW, F)
           .transpose(0, 1, 3, 2).reshape(NSPLIT, BH * F))
    idxs = jnp.concatenate(
        [idx + offs,
         jnp.full((B, SPAD - F), F * V, dtype=jnp.int32)], axis=1
    ).reshape(NSPLIT, BH * SPAD)

    table = W_second.reshape(F * V, E)
    wf = jnp.concatenate(
        [W_first.reshape(F * V), jnp.zeros((8,), jnp.float32)])

    mesh = plsc.VectorSubcoreMesh(
        core_axis_name="c", subcore_axis_name="s", num_cores=2)
    sc = functools.partial(
        pl.kernel,
        mesh=mesh,
        compiler_params=pltpu.CompilerParams(needs_layout_passes=False),
        out_type=(
            jax.ShapeDtypeStruct((BH, F * E), jnp.bfloat16),
            jax.ShapeDtypeStruct((BH * SPAD,), jnp.float32),
        ),
        scratch_types=[
            pltpu.VMEM((RPW,), jnp.int32),
            pltpu.VMEM((SPW * SPAD,), jnp.int32),
            pltpu.VMEM((F * V + 8,), jnp.float32),
            pltpu.VMEM((SPW * SPAD,), jnp.float32),
            pltpu.VMEM((RPW,), jnp.float32),
            pltpu.VMEM((SPW, E), jnp.float32),
            pltpu.VMEM((SPW, E), jnp.float32),
            pltpu.VMEM((SPW, E), jnp.bfloat16),
            pltpu.VMEM((SPW, E), jnp.bfloat16),
            pltpu.SemaphoreType.DMA,
            pltpu.SemaphoreType.DMA,
            pltpu.SemaphoreType.DMA,
            pltpu.SemaphoreType.DMA,
        ],
    )(_sc_gather)

    xvp = jnp.concatenate(
        [Xv, jnp.zeros((B, SPAD - F), jnp.float32)], axis=1)
    w1m = jnp.concatenate(
        [W1[1:1 + F], jnp.zeros((SPAD - F, D1), jnp.float32)],
        axis=0).astype(jnp.bfloat16)
    w1s = W1[1 + F:].astype(jnp.bfloat16)
    fmb = fm_bias.astype(jnp.bfloat16).astype(jnp.float32)
    b1eff = (b1[None, :]
             + fmb[:, None] * W1[0:1].astype(jnp.bfloat16).astype(jnp.float32))
    w2b = W2.astype(jnp.bfloat16)
    w3b = W3.astype(jnp.bfloat16)

    mlp = pl.pallas_call(
        _mlp_body,
        grid=(BH // BLK,),
        in_specs=[
            pl.BlockSpec((BLK, F * E), lambda i: (i, 0)),
            pl.BlockSpec((BLK, SPAD), lambda i: (i, 0)),
            pl.BlockSpec((BLK, SPAD), lambda i: (i, 0)),
            pl.BlockSpec((F * E, D1), lambda i: (0, 0)),
            pl.BlockSpec((SPAD, D1), lambda i: (0, 0)),
            pl.BlockSpec((1, D1), lambda i: (0, 0)),
            pl.BlockSpec((D1, D2), lambda i: (0, 0)),
            pl.BlockSpec((1, D2), lambda i: (0, 0)),
            pl.BlockSpec((D2, 1), lambda i: (0, 0)),
            pl.BlockSpec(memory_space=pltpu.SMEM),
        ],
        out_specs=pl.BlockSpec((BLK, 1), lambda i: (i, 0)),
        out_shape=jax.ShapeDtypeStruct((BH, 1), jnp.float32),
    )

    e2s, ss = [], []
    for h in range(NSPLIT):
        e2_h, s_h = sc(table, idx2[h], idxs[h], wf, xv2[h])
        e2s.append(e2_h)
        ss.append(s_h)
    outs = []
    for h in range(NSPLIT):
        out_h = mlp(e2s[h], xvp[h * BH:(h + 1) * BH],
                    ss[h].reshape(BH, SPAD), w1s, w1m, b1eff, w2b,
                    b2.reshape(1, D2), w3b, b3.reshape(1, 1))
        outs.append(out_h)
    return jnp.concatenate(outs, axis=0)


# speedup vs baseline: 1.0031x; 1.0031x over previous
import functools

import jax
import jax.numpy as jnp
from jax import lax
from jax.experimental import pallas as pl
from jax.experimental.pallas import tpu as pltpu
from jax.experimental.pallas import tpu_sc as plsc

B = 4096
F = 26
V = 1000
E = 128
D1 = 1024
D2 = 512

NSPLIT = 1
BH = B // NSPLIT
NUM_WORKERS = 32
SPW = BH // NUM_WORKERS
RPW = SPW * F
SPAD = 32
NBUF = 4
BLK = 512


def _sc_gather(table_hbm, idx2_hbm, idxs_hbm, wf_hbm, xv2_hbm,
               e2_out, s_out,
               idx_v, idxs_v, wf_v, s_v, xv_v,
               rows0, rows1, rows2, brows0, brows1,
               gsem0, gsem1, gsem2, ssem0, ssem1):
    wid = lax.axis_index("s") * 2 + lax.axis_index("c")
    base = wid * SPW
    rbase = wid * RPW

    rows = (rows0, rows1, rows2)
    brows = (brows0, brows1)
    gsem = (gsem0, gsem1, gsem2)
    ssem = (ssem0, ssem1)
    gcp = [None, None, None]
    scp = [None, None]

    pltpu.sync_copy(idx2_hbm.at[pl.ds(rbase, RPW)], idx_v)

    def start_gather(c, p):
        idx_c = idx_v.at[pl.ds(c * SPW, SPW)]
        return pltpu.async_copy(table_hbm.at[idx_c], rows[p], gsem[p])

    gcp[0] = start_gather(0, 0)
    gcp[1] = start_gather(1, 1)

    pltpu.sync_copy(xv2_hbm.at[pl.ds(rbase, RPW)], xv_v)
    pltpu.sync_copy(idxs_hbm.at[pl.ds(base * SPAD, SPW * SPAD)], idxs_v)
    pltpu.sync_copy(wf_hbm, wf_v)

    def s_body(i, carry):
        ids = idxs_v[pl.ds(i * 16, 16)]
        vals = plsc.load_gather(wf_v, [ids])
        s_v[pl.ds(i * 16, 16)] = vals
        return carry

    lax.fori_loop(0, SPW * SPAD // 16, s_body, 0)
    pltpu.sync_copy(s_v, s_out.at[pl.ds(base * SPAD, SPW * SPAD)])

    for c in range(F):
        if c + 2 < F:
            gcp[(c + 2) % 3] = start_gather(c + 2, (c + 2) % 3)
        gcp[c % 3].wait()
        buf = rows[c % 3]
        bbuf = brows[c % 2]
        if scp[c % 2] is not None:
            scp[c % 2].wait()

        def scale_row(r, carry, buf=buf, bbuf=bbuf, c=c):
            bc = plsc.load_gather(
                xv_v, [jnp.full((16,), c * SPW + r, jnp.int32)])
            for k in range(E // 32):
                v0 = buf[r, pl.ds(k * 32, 16)] * bc
                v1 = buf[r, pl.ds(k * 32 + 16, 16)] * bc
                bbuf[r, pl.ds(k * 32, 32)] = plsc.pack(
                    v0, v1, format=plsc.PackFormat.INTERLEAVED)
            return carry

        lax.fori_loop(0, SPW, scale_row, 0)
        scp[c % 2] = pltpu.async_copy(
            bbuf, e2_out.at[pl.ds(base, SPW), pl.ds(c * E, E)], ssem[c % 2])
    for p in range(2):
        if scp[p] is not None:
            scp[p].wait()


def _mlp_body(e2_ref, xv_ref, s_ref, w1s_ref, w1m_ref,
              b1_ref, w2_ref, b2_ref, w3_ref, b3_ref, out_ref):
    scaled = e2_ref[:]
    acc = jnp.dot(scaled, w1s_ref[:], preferred_element_type=jnp.float32)
    acc += jnp.dot((s_ref[:] * xv_ref[:]).astype(jnp.bfloat16), w1m_ref[:],
                   preferred_element_type=jnp.float32)
    h1 = jnp.tanh(acc + b1_ref[:])
    h2 = jnp.tanh(jnp.dot(h1.astype(jnp.bfloat16), w2_ref[:],
                          preferred_element_type=jnp.float32) + b2_ref[:])
    out_ref[:] = (jnp.dot(h2.astype(jnp.bfloat16), w3_ref[:],
                          preferred_element_type=jnp.float32) + b3_ref[0, 0])


def kernel(Xi, Xv, fm_bias, W_first, W_second, W1, b1, W2, b2, W3, b3):
    idx = Xi[:, :, 0].astype(jnp.int32)
    offs = (jnp.arange(F, dtype=jnp.int32) * V)[None, :]
    idx2 = ((idx + offs).reshape(NSPLIT, NUM_WORKERS, SPW, F)
            .transpose(0, 1, 3, 2).reshape(NSPLIT, BH * F))
    xv2 = (Xv.reshape(NSPLIT, NUM_WORKERS, SPW, F)
           .transpose(0, 1, 3, 2).reshape(NSPLIT, BH * F))
    idxs = jnp.concatenate(
        [idx + offs,
         jnp.full((B, SPAD - F), F * V, dtype=jnp.int32)], axis=1
    ).reshape(NSPLIT, BH * SPAD)

    table = W_second.reshape(F * V, E)
    wf = jnp.concatenate(
        [W_first.reshape(F * V), jnp.zeros((8,), jnp.float32)])

    mesh = plsc.VectorSubcoreMesh(
        core_axis_name="c", subcore_axis_name="s", num_cores=2)
    sc = functools.partial(
        pl.kernel,
        mesh=mesh,
        compiler_params=pltpu.CompilerParams(needs_layout_passes=False),
        out_type=(
            jax.ShapeDtypeStruct((BH, F * E), jnp.bfloat16),
            jax.ShapeDtypeStruct((BH * SPAD,), jnp.float32),
        ),
        scratch_types=[
            pltpu.VMEM((RPW,), jnp.int32),
            pltpu.VMEM((SPW * SPAD,), jnp.int32),
            pltpu.VMEM((F * V + 8,), jnp.float32),
            pltpu.VMEM((SPW * SPAD,), jnp.float32),
            pltpu.VMEM((RPW,), jnp.float32),
            pltpu.VMEM((SPW, E), jnp.float32),
            pltpu.VMEM((SPW, E), jnp.float32),
            pltpu.VMEM((SPW, E), jnp.float32),
            pltpu.VMEM((SPW, E), jnp.bfloat16),
            pltpu.VMEM((SPW, E), jnp.bfloat16),
            pltpu.SemaphoreType.DMA,
            pltpu.SemaphoreType.DMA,
            pltpu.SemaphoreType.DMA,
            pltpu.SemaphoreType.DMA,
            pltpu.SemaphoreType.DMA,
        ],
    )(_sc_gather)

    xvp = jnp.concatenate(
        [Xv, jnp.zeros((B, SPAD - F), jnp.float32)], axis=1)
    w1m = jnp.concatenate(
        [W1[1:1 + F], jnp.zeros((SPAD - F, D1), jnp.float32)],
        axis=0).astype(jnp.bfloat16)
    w1s = W1[1 + F:].astype(jnp.bfloat16)
    fmb = fm_bias.astype(jnp.bfloat16).astype(jnp.float32)
    b1eff = (b1[None, :]
             + fmb[:, None] * W1[0:1].astype(jnp.bfloat16).astype(jnp.float32))
    w2b = W2.astype(jnp.bfloat16)
    w3b = W3.astype(jnp.bfloat16)

    mlp = pl.pallas_call(
        _mlp_body,
        grid=(BH // BLK,),
        in_specs=[
            pl.BlockSpec((BLK, F * E), lambda i: (i, 0)),
            pl.BlockSpec((BLK, SPAD), lambda i: (i, 0)),
            pl.BlockSpec((BLK, SPAD), lambda i: (i, 0)),
            pl.BlockSpec((F * E, D1), lambda i: (0, 0)),
            pl.BlockSpec((SPAD, D1), lambda i: (0, 0)),
            pl.BlockSpec((1, D1), lambda i: (0, 0)),
            pl.BlockSpec((D1, D2), lambda i: (0, 0)),
            pl.BlockSpec((1, D2), lambda i: (0, 0)),
            pl.BlockSpec((D2, 1), lambda i: (0, 0)),
            pl.BlockSpec(memory_space=pltpu.SMEM),
        ],
        out_specs=pl.BlockSpec((BLK, 1), lambda i: (i, 0)),
        out_shape=jax.ShapeDtypeStruct((BH, 1), jnp.float32),
    )

    e2s, ss = [], []
    for h in range(NSPLIT):
        e2_h, s_h = sc(table, idx2[h], idxs[h], wf, xv2[h])
        e2s.append(e2_h)
        ss.append(s_h)
    outs = []
    for h in range(NSPLIT):
        out_h = mlp(e2s[h], xvp[h * BH:(h + 1) * BH],
                    ss[h].reshape(BH, SPAD), w1s, w1m, b1eff, w2b,
                    b2.reshape(1, D2), w3b, b3.reshape(1, 1))
        outs.append(out_h)
    return jnp.concatenate(outs, axis=0)


# speedup vs baseline: 1.0360x; 1.0328x over previous
import functools

import jax
import jax.numpy as jnp
from jax import lax
from jax.experimental import pallas as pl
from jax.experimental.pallas import tpu as pltpu
from jax.experimental.pallas import tpu_sc as plsc

B = 4096
F = 26
V = 1000
E = 128
D1 = 1024
D2 = 512

NSPLIT = 1
BH = B // NSPLIT
NUM_WORKERS = 32
SPW = BH // NUM_WORKERS
RPW = SPW * F
SPAD = 32
NBUF = 4
BLK = 512


def _sc_gather(table_hbm, idx2_hbm, idxs_hbm, wf_hbm, xv2_hbm,
               e2_out, s_out,
               idx_v, idxs_v, wf_v, s_v, xv_v,
               rows0, rows1, rows2, brows0, brows1,
               gsem0, gsem1, gsem2, ssem0, ssem1):
    wid = lax.axis_index("s") * 2 + lax.axis_index("c")
    base = wid * SPW
    rbase = wid * RPW

    rows = (rows0, rows1, rows2)
    brows = (brows0, brows1)
    gsem = (gsem0, gsem1, gsem2)
    ssem = (ssem0, ssem1)
    gcp = [None, None, None]
    scp = [None, None]

    pltpu.sync_copy(idx2_hbm.at[pl.ds(rbase, RPW)], idx_v)

    def start_gather(c, p):
        idx_c = idx_v.at[pl.ds(c * SPW, SPW)]
        return pltpu.async_copy(table_hbm.at[idx_c], rows[p], gsem[p])

    gcp[0] = start_gather(0, 0)
    gcp[1] = start_gather(1, 1)

    pltpu.sync_copy(xv2_hbm.at[pl.ds(rbase, RPW)], xv_v)
    pltpu.sync_copy(idxs_hbm.at[pl.ds(base * SPAD, SPW * SPAD)], idxs_v)
    pltpu.sync_copy(wf_hbm, wf_v)

    def s_body(i, carry):
        ids = idxs_v[pl.ds(i * 16, 16)]
        vals = plsc.load_gather(wf_v, [ids])
        s_v[pl.ds(i * 16, 16)] = vals
        return carry

    lax.fori_loop(0, SPW * SPAD // 16, s_body, 0)
    pltpu.sync_copy(s_v, s_out.at[pl.ds(base * SPAD, SPW * SPAD)])

    for c in range(F):
        if c + 2 < F:
            gcp[(c + 2) % 3] = start_gather(c + 2, (c + 2) % 3)
        gcp[c % 3].wait()
        buf = rows[c % 3]
        bbuf = brows[c % 2]
        if scp[c % 2] is not None:
            scp[c % 2].wait()

        def scale_row(i, carry, buf=buf, bbuf=bbuf, c=c):
            for off in (0, SPW // 2):
                r = i + off
                bc = plsc.load_gather(
                    xv_v, [jnp.full((16,), c * SPW + r, jnp.int32)])
                for k in range(E // 32):
                    v0 = buf[r, pl.ds(k * 32, 16)] * bc
                    v1 = buf[r, pl.ds(k * 32 + 16, 16)] * bc
                    bbuf[r, pl.ds(k * 32, 32)] = plsc.pack(
                        v0, v1, format=plsc.PackFormat.INTERLEAVED)
            return carry

        lax.fori_loop(0, SPW // 2, scale_row, 0)
        scp[c % 2] = pltpu.async_copy(
            bbuf, e2_out.at[pl.ds(base, SPW), pl.ds(c * E, E)], ssem[c % 2])
    for p in range(2):
        if scp[p] is not None:
            scp[p].wait()


def _mlp_body(e2_ref, xv_ref, s_ref, w1s_ref, w1m_ref,
              b1_ref, w2_ref, b2_ref, w3_ref, b3_ref, out_ref):
    scaled = e2_ref[:]
    acc = jnp.dot(scaled, w1s_ref[:], preferred_element_type=jnp.float32)
    acc += jnp.dot((s_ref[:] * xv_ref[:]).astype(jnp.bfloat16), w1m_ref[:],
                   preferred_element_type=jnp.float32)
    h1 = jnp.tanh(acc + b1_ref[:])
    h2 = jnp.tanh(jnp.dot(h1.astype(jnp.bfloat16), w2_ref[:],
                          preferred_element_type=jnp.float32) + b2_ref[:])
    out_ref[:] = (jnp.dot(h2.astype(jnp.bfloat16), w3_ref[:],
                          preferred_element_type=jnp.float32) + b3_ref[0, 0])


def kernel(Xi, Xv, fm_bias, W_first, W_second, W1, b1, W2, b2, W3, b3):
    idx = Xi[:, :, 0].astype(jnp.int32)
    offs = (jnp.arange(F, dtype=jnp.int32) * V)[None, :]
    idx2 = ((idx + offs).reshape(NSPLIT, NUM_WORKERS, SPW, F)
            .transpose(0, 1, 3, 2).reshape(NSPLIT, BH * F))
    xv2 = (Xv.reshape(NSPLIT, NUM_WORKERS, SPW, F)
           .transpose(0, 1, 3, 2).reshape(NSPLIT, BH * F))
    idxs = jnp.concatenate(
        [idx + offs,
         jnp.full((B, SPAD - F), F * V, dtype=jnp.int32)], axis=1
    ).reshape(NSPLIT, BH * SPAD)

    table = W_second.reshape(F * V, E)
    wf = jnp.concatenate(
        [W_first.reshape(F * V), jnp.zeros((8,), jnp.float32)])

    mesh = plsc.VectorSubcoreMesh(
        core_axis_name="c", subcore_axis_name="s", num_cores=2)
    sc = functools.partial(
        pl.kernel,
        mesh=mesh,
        compiler_params=pltpu.CompilerParams(needs_layout_passes=False),
        out_type=(
            jax.ShapeDtypeStruct((BH, F * E), jnp.bfloat16),
            jax.ShapeDtypeStruct((BH * SPAD,), jnp.float32),
        ),
        scratch_types=[
            pltpu.VMEM((RPW,), jnp.int32),
            pltpu.VMEM((SPW * SPAD,), jnp.int32),
            pltpu.VMEM((F * V + 8,), jnp.float32),
            pltpu.VMEM((SPW * SPAD,), jnp.float32),
            pltpu.VMEM((RPW,), jnp.float32),
            pltpu.VMEM((SPW, E), jnp.float32),
            pltpu.VMEM((SPW, E), jnp.float32),
            pltpu.VMEM((SPW, E), jnp.float32),
            pltpu.VMEM((SPW, E), jnp.bfloat16),
            pltpu.VMEM((SPW, E), jnp.bfloat16),
            pltpu.SemaphoreType.DMA,
            pltpu.SemaphoreType.DMA,
            pltpu.SemaphoreType.DMA,
            pltpu.SemaphoreType.DMA,
            pltpu.SemaphoreType.DMA,
        ],
    )(_sc_gather)

    xvp = jnp.concatenate(
        [Xv, jnp.zeros((B, SPAD - F), jnp.float32)], axis=1)
    w1m = jnp.concatenate(
        [W1[1:1 + F], jnp.zeros((SPAD - F, D1), jnp.float32)],
        axis=0).astype(jnp.bfloat16)
    w1s = W1[1 + F:].astype(jnp.bfloat16)
    fmb = fm_bias.astype(jnp.bfloat16).astype(jnp.float32)
    b1eff = (b1[None, :]
             + fmb[:, None] * W1[0:1].astype(jnp.bfloat16).astype(jnp.float32))
    w2b = W2.astype(jnp.bfloat16)
    w3b = W3.astype(jnp.bfloat16)

    mlp = pl.pallas_call(
        _mlp_body,
        grid=(BH // BLK,),
        in_specs=[
            pl.BlockSpec((BLK, F * E), lambda i: (i, 0)),
            pl.BlockSpec((BLK, SPAD), lambda i: (i, 0)),
            pl.BlockSpec((BLK, SPAD), lambda i: (i, 0)),
            pl.BlockSpec((F * E, D1), lambda i: (0, 0)),
            pl.BlockSpec((SPAD, D1), lambda i: (0, 0)),
            pl.BlockSpec((1, D1), lambda i: (0, 0)),
            pl.BlockSpec((D1, D2), lambda i: (0, 0)),
            pl.BlockSpec((1, D2), lambda i: (0, 0)),
            pl.BlockSpec((D2, 1), lambda i: (0, 0)),
            pl.BlockSpec(memory_space=pltpu.SMEM),
        ],
        out_specs=pl.BlockSpec((BLK, 1), lambda i: (i, 0)),
        out_shape=jax.ShapeDtypeStruct((BH, 1), jnp.float32),
    )

    e2s, ss = [], []
    for h in range(NSPLIT):
        e2_h, s_h = sc(table, idx2[h], idxs[h], wf, xv2[h])
        e2s.append(e2_h)
        ss.append(s_h)
    outs = []
    for h in range(NSPLIT):
        out_h = mlp(e2s[h], xvp[h * BH:(h + 1) * BH],
                    ss[h].reshape(BH, SPAD), w1s, w1m, b1eff, w2b,
                    b2.reshape(1, D2), w3b, b3.reshape(1, 1))
        outs.append(out_h)
    return jnp.concatenate(outs, axis=0)


# speedup vs baseline: 1.2953x; 1.2503x over previous
import functools

import jax
import jax.numpy as jnp
from jax import lax
from jax.experimental import pallas as pl
from jax.experimental.pallas import tpu as pltpu
from jax.experimental.pallas import tpu_sc as plsc

B = 4096
F = 26
V = 1000
E = 128
D1 = 1024
D2 = 512

NSPLIT = 1
BH = B // NSPLIT
NUM_WORKERS = 32
SPW = BH // NUM_WORKERS
RPW = SPW * F
SPAD = 32
NBUF = 4
BLK = 512


def _sc_gather(table_hbm, idx2_hbm, idxs_hbm, wf_hbm, xv2_hbm,
               e2_out, s_out,
               idx_v, idxs_v, wf_v, s_v, xv_v,
               rows0, rows1, rows2, rows3,
               gsem0, gsem1, gsem2, gsem3,
               ssem0, ssem1, ssem2, ssem3):
    wid = lax.axis_index("s") * 2 + lax.axis_index("c")
    base = wid * SPW
    rbase = wid * RPW

    rows = (rows0, rows1, rows2, rows3)
    gsem = (gsem0, gsem1, gsem2, gsem3)
    ssem = (ssem0, ssem1, ssem2, ssem3)
    gcp = [None] * NBUF
    scp = [None] * NBUF

    pltpu.sync_copy(idx2_hbm.at[pl.ds(rbase, RPW)], idx_v)

    def start_gather(c, p):
        idx_c = idx_v.at[pl.ds(c * SPW, SPW)]
        return pltpu.async_copy(table_hbm.at[idx_c], rows[p], gsem[p])

    gcp[0] = start_gather(0, 0)
    gcp[1] = start_gather(1, 1)

    pltpu.sync_copy(xv2_hbm.at[pl.ds(rbase, RPW)], xv_v)
    pltpu.sync_copy(idxs_hbm.at[pl.ds(base * SPAD, SPW * SPAD)], idxs_v)
    pltpu.sync_copy(wf_hbm, wf_v)

    def s_body(i, carry):
        ids = idxs_v[pl.ds(i * 16, 16)]
        vals = plsc.load_gather(wf_v, [ids])
        s_v[pl.ds(i * 16, 16)] = vals
        return carry

    lax.fori_loop(0, SPW * SPAD // 16, s_body, 0)
    pltpu.sync_copy(s_v, s_out.at[pl.ds(base * SPAD, SPW * SPAD)])

    for c in range(F):
        p = c % NBUF
        if c + 2 < F:
            q = (c + 2) % NBUF
            if scp[q] is not None:
                scp[q].wait()
            gcp[q] = start_gather(c + 2, q)
        gcp[p].wait()

        buf = rows[p]

        def scale_row(r, carry, buf=buf, c=c):
            bc = plsc.load_gather(
                xv_v, [jnp.full((16,), c * SPW + r, jnp.int32)])
            for k in range(E // 16):
                buf[r, pl.ds(k * 16, 16)] = buf[r, pl.ds(k * 16, 16)] * bc
            return carry

        lax.fori_loop(0, SPW, scale_row, 0)
        scp[p] = pltpu.async_copy(
            buf, e2_out.at[pl.ds(base, SPW), pl.ds(c * E, E)], ssem[p])
    for p in range(NBUF):
        if scp[p] is not None:
            scp[p].wait()


def _mlp_body(e2_ref, xv_ref, s_ref, w1s_ref, w1m_ref,
              b1_ref, w2_ref, b2_ref, w3_ref, b3_ref, out_ref):
    scaled = e2_ref[:].astype(jnp.bfloat16)
    acc = jnp.dot(scaled, w1s_ref[:], preferred_element_type=jnp.float32)
    acc += jnp.dot((s_ref[:] * xv_ref[:]).astype(jnp.bfloat16), w1m_ref[:],
                   preferred_element_type=jnp.float32)
    h1 = jnp.tanh(acc + b1_ref[:])
    h2 = jnp.tanh(jnp.dot(h1.astype(jnp.bfloat16), w2_ref[:],
                          preferred_element_type=jnp.float32) + b2_ref[:])
    out_ref[:] = (jnp.dot(h2.astype(jnp.bfloat16), w3_ref[:],
                          preferred_element_type=jnp.float32) + b3_ref[0, 0])


def kernel(Xi, Xv, fm_bias, W_first, W_second, W1, b1, W2, b2, W3, b3):
    idx = Xi[:, :, 0].astype(jnp.int32)
    offs = (jnp.arange(F, dtype=jnp.int32) * V)[None, :]
    idx2 = ((idx + offs).reshape(NSPLIT, NUM_WORKERS, SPW, F)
            .transpose(0, 1, 3, 2).reshape(NSPLIT, BH * F))
    xv2 = (Xv.reshape(NSPLIT, NUM_WORKERS, SPW, F)
           .transpose(0, 1, 3, 2).reshape(NSPLIT, BH * F))
    idxs = jnp.concatenate(
        [idx + offs,
         jnp.full((B, SPAD - F), F * V, dtype=jnp.int32)], axis=1
    ).reshape(NSPLIT, BH * SPAD)

    table = W_second.reshape(F * V, E)
    wf = jnp.concatenate(
        [W_first.reshape(F * V), jnp.zeros((8,), jnp.float32)])

    mesh = plsc.VectorSubcoreMesh(
        core_axis_name="c", subcore_axis_name="s", num_cores=2)
    sc = functools.partial(
        pl.kernel,
        mesh=mesh,
        compiler_params=pltpu.CompilerParams(needs_layout_passes=False),
        out_type=(
            jax.ShapeDtypeStruct((BH, F * E), jnp.float32),
            jax.ShapeDtypeStruct((BH * SPAD,), jnp.float32),
        ),
        scratch_types=[
            pltpu.VMEM((RPW,), jnp.int32),
            pltpu.VMEM((SPW * SPAD,), jnp.int32),
            pltpu.VMEM((F * V + 8,), jnp.float32),
            pltpu.VMEM((SPW * SPAD,), jnp.float32),
            pltpu.VMEM((RPW,), jnp.float32),
            pltpu.VMEM((SPW, E), jnp.float32),
            pltpu.VMEM((SPW, E), jnp.float32),
            pltpu.VMEM((SPW, E), jnp.float32),
            pltpu.VMEM((SPW, E), jnp.float32),
            pltpu.SemaphoreType.DMA,
            pltpu.SemaphoreType.DMA,
            pltpu.SemaphoreType.DMA,
            pltpu.SemaphoreType.DMA,
            pltpu.SemaphoreType.DMA,
            pltpu.SemaphoreType.DMA,
            pltpu.SemaphoreType.DMA,
            pltpu.SemaphoreType.DMA,
        ],
    )(_sc_gather)

    xvp = jnp.concatenate(
        [Xv, jnp.zeros((B, SPAD - F), jnp.float32)], axis=1)
    w1m = jnp.concatenate(
        [W1[1:1 + F], jnp.zeros((SPAD - F, D1), jnp.float32)],
        axis=0).astype(jnp.bfloat16)
    w1s = W1[1 + F:].astype(jnp.bfloat16)
    fmb = fm_bias.astype(jnp.bfloat16).astype(jnp.float32)
    b1eff = (b1[None, :]
             + fmb[:, None] * W1[0:1].astype(jnp.bfloat16).astype(jnp.float32))
    w2b = W2.astype(jnp.bfloat16)
    w3b = W3.astype(jnp.bfloat16)

    mlp = pl.pallas_call(
        _mlp_body,
        grid=(BH // BLK,),
        in_specs=[
            pl.BlockSpec((BLK, F * E), lambda i: (i, 0)),
            pl.BlockSpec((BLK, SPAD), lambda i: (i, 0)),
            pl.BlockSpec((BLK, SPAD), lambda i: (i, 0)),
            pl.BlockSpec((F * E, D1), lambda i: (0, 0)),
            pl.BlockSpec((SPAD, D1), lambda i: (0, 0)),
            pl.BlockSpec((1, D1), lambda i: (0, 0)),
            pl.BlockSpec((D1, D2), lambda i: (0, 0)),
            pl.BlockSpec((1, D2), lambda i: (0, 0)),
            pl.BlockSpec((D2, 1), lambda i: (0, 0)),
            pl.BlockSpec(memory_space=pltpu.SMEM),
        ],
        out_specs=pl.BlockSpec((BLK, 1), lambda i: (i, 0)),
        out_shape=jax.ShapeDtypeStruct((BH, 1), jnp.float32),
    )

    e2s, ss = [], []
    for h in range(NSPLIT):
        e2_h, s_h = sc(table, idx2[h], idxs[h], wf, xv2[h])
        e2s.append(e2_h)
        ss.append(s_h)
    outs = []
    for h in range(NSPLIT):
        out_h = mlp(e2s[h], xvp[h * BH:(h + 1) * BH],
                    ss[h].reshape(BH, SPAD), w1s, w1m, b1eff, w2b,
                    b2.reshape(1, D2), w3b, b3.reshape(1, 1))
        outs.append(out_h)
    return jnp.concatenate(outs, axis=0)
